# baseline (speedup 1.0000x reference)
import jax
import jax.numpy as jnp
from jax import lax
from jax.experimental import pallas as pl
from jax.experimental.pallas import tpu as pltpu
from jax.experimental.pallas import tpu_sc as plsc

_K = 9
_NEG_INF = float("-inf")
_SC_CORES = 2
_SC_SUBCORES = 16
_NODES_PER_WORKER = 128
_CHUNK = 16


def _tc1_body(xt_ref, w_ref, idx_ref, yut_ref, yv_ref):
    b = pl.program_id(0)
    xt = xt_ref[0]
    n = xt.shape[0]
    cg = xt.shape[1] // 2

    nrm = jnp.sqrt(jnp.sum(xt * xt, axis=1, keepdims=True))
    xn = xt / jnp.maximum(nrm, 1e-12)
    x_sq = jnp.sum(xn * xn, axis=1, keepdims=True)
    inner = lax.dot_general(xn, xn, (((1,), (1,)), ((), ())),
                            preferred_element_type=jnp.float32)
    neg_dist = 2.0 * inner - x_sq - jnp.transpose(x_sq)

    w0 = w_ref[0:cg]
    w1 = w_ref[cg:2 * cg]
    w2 = w_ref[2 * cg:3 * cg]
    w3 = w_ref[3 * cg:4 * cg]
    xa = xt[:, :cg]
    xb = xt[:, cg:]

    def mm_t(wb, xp):
        return lax.dot_general(wb, xp, (((1,), (1,)), ((), ())),
                               preferred_element_type=jnp.float32)

    def mm(xp, wb):
        return lax.dot_general(xp, wb, (((1,), (1,)), ((), ())),
                               preferred_element_type=jnp.float32)

    yut_ref[0] = jnp.concatenate([mm_t(w0, xa), mm_t(w1, xb)], axis=0)
    yv_ref[0] = jnp.concatenate([mm(xa, w2), mm(xb, w3)], axis=1)

    row = lax.broadcasted_iota(jnp.int32, (n, n), 0)
    base = b * n

    def body(k, nd):
        m = jnp.max(nd, axis=0, keepdims=True)
        sel = jnp.min(jnp.where(nd == m, row, n), axis=0,
                      keepdims=True)
        idx_ref[0, pl.ds(k, 1), :] = sel + base
        return jnp.where(row == sel, _NEG_INF, nd)

    lax.fori_loop(0, _K, body, neg_dist)


def _sc_body(yv_hbm, idx_hbm, agg_hbm, idx_v, rows_v, out_v, sem):
    wid = lax.axis_index("s") * _SC_CORES + lax.axis_index("c")
    npw = _NODES_PER_WORKER
    b = wid // 8
    nbase = (wid % 8) * npw
    pltpu.sync_copy(idx_hbm.at[b, :, pl.ds(nbase, npw)], idx_v)

    @pl.loop(0, npw // _CHUNK)
    def _chunk_loop(chunk):
        cb = chunk * _CHUNK
        copies = []
        for k in range(_K):
            copies.append(pltpu.async_copy(
                yv_hbm.at[idx_v.at[k, pl.ds(cb, _CHUNK)]],
                rows_v.at[k], sem))
        for c in copies:
            c.wait()

        @pl.loop(0, _CHUNK)
        def _node_loop(i):
            @pl.loop(0, 384, step=16)
            def _lane_loop(c0):
                acc = rows_v.at[0, i, pl.ds(c0, 16)][...]
                for k in range(1, _K):
                    acc = jnp.maximum(acc,
                                      rows_v.at[k, i, pl.ds(c0, 16)][...])
                out_v.at[i, pl.ds(c0, 16)][...] = acc

        pltpu.sync_copy(
            out_v, agg_hbm.at[pl.ds(b * 1024 + nbase + cb, _CHUNK)])


def _tc2_body(agg_ref, yv_ref, yut_ref, b_ref, out_ref):
    half = yut_ref.shape[1]
    d = agg_ref[0] - yv_ref[0]
    dt = jnp.transpose(d)
    out_ref[0, 0:half, :] = jnp.maximum(yut_ref[0] + b_ref[0:half], 0.0)
    out_ref[0, half:, :] = jnp.maximum(dt + b_ref[half:], 0.0)


def _sc_gather_max(yv_flat, idx):
    n_rows = yv_flat.shape[0]
    f = pl.kernel(
        _sc_body,
        out_type=jax.ShapeDtypeStruct((n_rows, yv_flat.shape[1]),
                                      jnp.float32),
        mesh=plsc.VectorSubcoreMesh(core_axis_name="c",
                                    subcore_axis_name="s"),
        scratch_types=[
            pltpu.VMEM((_K, _NODES_PER_WORKER), jnp.int32),
            pltpu.VMEM((_K, _CHUNK, 384), jnp.float32),
            pltpu.VMEM((_CHUNK, 384), jnp.float32),
            pltpu.SemaphoreType.DMA,
        ],
    )
    return f(yv_flat, idx)


@jax.jit
def kernel(x, conv_w, conv_b):
    B, C, H, W = x.shape
    N = H * W
    Cout = conv_w.shape[0]
    half = Cout // 2
    xt = jnp.transpose(x.reshape(B, C, N), (0, 2, 1))

    idx, yut, yv = pl.pallas_call(
        _tc1_body,
        grid=(B,),
        in_specs=[
            pl.BlockSpec((1, N, C), lambda b: (b, 0, 0)),
            pl.BlockSpec((Cout, conv_w.shape[1]), lambda b: (0, 0)),
        ],
        out_specs=[
            pl.BlockSpec((1, _K, N), lambda b: (b, 0, 0)),
            pl.BlockSpec((1, half, N), lambda b: (b, 0, 0)),
            pl.BlockSpec((1, N, half), lambda b: (b, 0, 0)),
        ],
        out_shape=[
            jax.ShapeDtypeStruct((B, _K, N), jnp.int32),
            jax.ShapeDtypeStruct((B, half, N), jnp.float32),
            jax.ShapeDtypeStruct((B, N, half), jnp.float32),
        ],
    )(xt, conv_w)

    agg = _sc_gather_max(yv.reshape(B * N, half), idx)

    out = pl.pallas_call(
        _tc2_body,
        grid=(B,),
        in_specs=[
            pl.BlockSpec((1, N, half), lambda b: (b, 0, 0)),
            pl.BlockSpec((1, N, half), lambda b: (b, 0, 0)),
            pl.BlockSpec((1, half, N), lambda b: (b, 0, 0)),
            pl.BlockSpec((Cout, 1), lambda b: (0, 0)),
        ],
        out_specs=pl.BlockSpec((1, Cout, N), lambda b: (b, 0, 0)),
        out_shape=jax.ShapeDtypeStruct((B, Cout, N), jnp.float32),
    )(agg.reshape(B, N, half), yv, yut, conv_b.reshape(Cout, 1))

    return out.reshape(B, Cout, H, W)


# speedup vs baseline: 1.1501x; 1.1501x over previous
import jax
import jax.numpy as jnp
from jax import lax
from jax.experimental import pallas as pl
from jax.experimental.pallas import tpu as pltpu
from jax.experimental.pallas import tpu_sc as plsc

_K = 9
_NEG_INF = float("-inf")
_SC_CORES = 2
_NPW = 32
_CHUNK = 8


def _tc1_body(xt_ref, w_ref, idx_ref, yut_ref, yv_ref):
    xt = xt_ref[...]
    n = xt.shape[0]
    cg = xt.shape[1] // 2

    nrm = jnp.sqrt(jnp.sum(xt * xt, axis=1, keepdims=True))
    xn = xt / jnp.maximum(nrm, 1e-12)
    x_sq = jnp.sum(xn * xn, axis=1, keepdims=True)
    inner = lax.dot_general(xn, xn, (((1,), (1,)), ((), ())),
                            preferred_element_type=jnp.float32)
    neg_dist = 2.0 * inner - x_sq - jnp.transpose(x_sq)

    w0 = w_ref[0:cg]
    w1 = w_ref[cg:2 * cg]
    w2 = w_ref[2 * cg:3 * cg]
    w3 = w_ref[3 * cg:4 * cg]
    xa = xt[:, :cg]
    xb = xt[:, cg:]

    def mm_t(wb, xp):
        return lax.dot_general(wb, xp, (((1,), (1,)), ((), ())),
                               preferred_element_type=jnp.float32)

    def mm(xp, wb):
        return lax.dot_general(xp, wb, (((1,), (1,)), ((), ())),
                               preferred_element_type=jnp.float32)

    yut_ref[...] = jnp.concatenate([mm_t(w0, xa), mm_t(w1, xb)], axis=0)
    yv_ref[...] = jnp.concatenate([mm(xa, w2), mm(xb, w3)], axis=1)

    row = lax.broadcasted_iota(jnp.int32, (n, n), 0)

    def body(k, nd):
        m = jnp.max(nd, axis=0, keepdims=True)
        sel = jnp.min(jnp.where(nd == m, row, n), axis=0,
                      keepdims=True)
        idx_ref[pl.ds(k, 1), :] = sel
        return jnp.where(row == sel, _NEG_INF, nd)

    lax.fori_loop(0, _K, body, neg_dist)


def _sc_body(yv_hbm, idx_hbm, agg_hbm,
             idx_v, rows_a, rows_b, out_v, sem_a, sem_b):
    wid = lax.axis_index("s") * _SC_CORES + lax.axis_index("c")
    nbase = wid * _NPW
    pltpu.sync_copy(idx_hbm, idx_v)

    nchunks = _NPW // _CHUNK
    bufs = [(rows_a, sem_a), (rows_b, sem_b)]

    def fire(c):
        buf, sem = bufs[c % 2]
        return [pltpu.async_copy(
            yv_hbm.at[idx_v.at[k, pl.ds(nbase + c * _CHUNK, _CHUNK)]],
            buf.at[k], sem) for k in range(_K)]

    pending = fire(0)
    for c in range(nchunks):
        nxt = fire(c + 1) if c + 1 < nchunks else []
        for h in pending:
            h.wait()
        pending = nxt
        buf, _ = bufs[c % 2]

        @pl.loop(0, _CHUNK)
        def _node_loop(i):
            @pl.loop(0, 384, step=16)
            def _lane_loop(c0):
                acc = buf.at[0, i, pl.ds(c0, 16)][...]
                for k in range(1, _K):
                    acc = jnp.maximum(acc, buf.at[k, i, pl.ds(c0, 16)][...])
                out_v.at[i, pl.ds(c0, 16)][...] = acc

        pltpu.sync_copy(out_v,
                        agg_hbm.at[pl.ds(nbase + c * _CHUNK, _CHUNK)])


def _tc2_body(agg_ref, yv_ref, yut_ref, b_ref, out_ref):
    half = yut_ref.shape[0]
    d = agg_ref[...] - yv_ref[...]
    dt = jnp.transpose(d)
    out_ref[0:half, :] = jnp.maximum(yut_ref[...] + b_ref[0:half], 0.0)
    out_ref[half:, :] = jnp.maximum(dt + b_ref[half:], 0.0)


def _sc_gather_max(yv_b, idx_b):
    n, c = yv_b.shape
    f = pl.kernel(
        _sc_body,
        out_type=jax.ShapeDtypeStruct((n, c), jnp.float32),
        mesh=plsc.VectorSubcoreMesh(core_axis_name="c",
                                    subcore_axis_name="s"),
        scratch_types=[
            pltpu.VMEM((_K, 1024), jnp.int32),
            pltpu.VMEM((_K, _CHUNK, 384), jnp.float32),
            pltpu.VMEM((_K, _CHUNK, 384), jnp.float32),
            pltpu.VMEM((_CHUNK, 384), jnp.float32),
            pltpu.SemaphoreType.DMA,
            pltpu.SemaphoreType.DMA,
        ],
    )
    return f(yv_b, idx_b)


@jax.jit
def kernel(x, conv_w, conv_b):
    B, C, H, W = x.shape
    N = H * W
    Cout = conv_w.shape[0]
    half = Cout // 2
    xt = jnp.transpose(x.reshape(B, C, N), (0, 2, 1))
    bias_col = conv_b.reshape(Cout, 1)

    tc1 = pl.pallas_call(
        _tc1_body,
        out_shape=[
            jax.ShapeDtypeStruct((_K, N), jnp.int32),
            jax.ShapeDtypeStruct((half, N), jnp.float32),
            jax.ShapeDtypeStruct((N, half), jnp.float32),
        ],
    )

    tc2 = pl.pallas_call(
        _tc2_body,
        out_shape=jax.ShapeDtypeStruct((Cout, N), jnp.float32),
    )

    outs = []
    for b in range(B):
        idx_b, yut_b, yv_b = tc1(xt[b], conv_w)
        agg_b = _sc_gather_max(yv_b, idx_b)
        outs.append(tc2(agg_b, yv_b, yut_b, bias_col))

    return jnp.stack(outs).reshape(B, Cout, H, W)
